# Initial kernel scaffold; baseline (speedup 1.0000x reference)
import jax
import jax.numpy as jnp
from jax.experimental import pallas as pl


def kernel(x, table):
    raise NotImplementedError("write your pallas kernel here")


# speedup vs baseline: 1.1027x; 1.1027x over previous
import functools
import jax
import jax.numpy as jnp
from jax import lax
from jax.experimental import pallas as pl
from jax.experimental.pallas import tpu as pltpu
from jax.experimental.pallas import tpu_sc as plsc

NUM_EMB = 1000000
DIM = 32
TOTAL = 16384 * 50

_info = plsc.get_sparse_core_info()
NC, NS = _info.num_cores, _info.num_subcores
NW = NC * NS
B_PER_W = TOTAL // NW
G = 128
NG = B_PER_W // G
K = 8
NGROUP = NG // K
GROUP_ROWS = K * G


def _body(table_hbm, idx_hbm, out_hbm, idx_v, rows_v, sem):
    wid = lax.axis_index("s") * NC + lax.axis_index("c")
    base = wid * B_PER_W

    pltpu.sync_copy(idx_hbm.at[pl.ds(wid * NG, NG)], idx_v)

    def group(g, _):
        row0 = base + g * GROUP_ROWS
        descs = []
        for k in range(K):
            d = pltpu.async_copy(
                table_hbm.at[idx_v.at[g * K + k]],
                rows_v.at[pl.ds(k * G, G)],
                sem,
            )
            descs.append(d)
        for d in descs:
            d.wait()
        pltpu.sync_copy(rows_v, out_hbm.at[pl.ds(row0, GROUP_ROWS)])
        return ()

    lax.fori_loop(0, NGROUP, group, ())


@jax.jit
def _gather_flat(table, idx2d):
    mesh = plsc.VectorSubcoreMesh(core_axis_name="c", subcore_axis_name="s")
    return pl.kernel(
        _body,
        out_type=jax.ShapeDtypeStruct((TOTAL, DIM), jnp.float32),
        mesh=mesh,
        scratch_types=[
            pltpu.VMEM((NG, G), jnp.int32),
            pltpu.VMEM((GROUP_ROWS, DIM), jnp.float32),
            pltpu.SemaphoreType.DMA,
        ],
        compiler_params=pltpu.CompilerParams(use_tc_tiling_on_sc=False),
    )(table, idx2d)


def kernel(x, table):
    idx2d = x.reshape(TOTAL // G, G).astype(jnp.int32)
    out = _gather_flat(table, idx2d)
    return out.reshape(x.shape[0], x.shape[1], DIM)


# speedup vs baseline: 1.1102x; 1.0067x over previous
import jax
import jax.numpy as jnp
from jax import lax
from jax.experimental import pallas as pl
from jax.experimental.pallas import tpu as pltpu
from jax.experimental.pallas import tpu_sc as plsc

NUM_EMB = 1000000
DIM = 32
TOTAL = 16384 * 50

_info = plsc.get_sparse_core_info()
NC, NS = _info.num_cores, _info.num_subcores
NW = NC * NS
B_PER_W = TOTAL // NW
G = 128
NG = B_PER_W // G
K = 10
GROUP_ROWS = K * G
NGROUP = NG // K
NITER = NGROUP // 2


def _body(table_hbm, idx_hbm, out_hbm, idx_v, rows0, rows1,
          gsem0, gsem1, osem0, osem1):
    wid = lax.axis_index("s") * NC + lax.axis_index("c")
    base = wid * B_PER_W

    pltpu.sync_copy(idx_hbm.at[pl.ds(wid * NG, NG)], idx_v)

    def fire(g, buf, sem):
        for k in range(K):
            pltpu.async_copy(
                table_hbm.at[idx_v.at[g * K + k]],
                buf.at[pl.ds(k * G, G)],
                sem,
            )

    def drain_gathers(buf, sem):
        pltpu.make_async_copy(out_hbm.at[pl.ds(0, GROUP_ROWS)], buf, sem).wait()

    fire(0, rows0, gsem0)

    def step(i, _):
        b = 2 * i

        @pl.when(i > 0)
        def _():
            pltpu.make_async_copy(
                rows1, out_hbm.at[pl.ds(0, GROUP_ROWS)], osem1).wait()

        fire(b + 1, rows1, gsem1)
        drain_gathers(rows0, gsem0)
        oc0 = pltpu.async_copy(
            rows0, out_hbm.at[pl.ds(base + b * GROUP_ROWS, GROUP_ROWS)], osem0)
        drain_gathers(rows1, gsem1)
        pltpu.async_copy(
            rows1, out_hbm.at[pl.ds(base + (b + 1) * GROUP_ROWS, GROUP_ROWS)],
            osem1)
        oc0.wait()

        @pl.when(i < NITER - 1)
        def _():
            fire(b + 2, rows0, gsem0)

        return ()

    lax.fori_loop(0, NITER, step, ())
    pltpu.make_async_copy(rows1, out_hbm.at[pl.ds(0, GROUP_ROWS)], osem1).wait()


@jax.jit
def _gather_flat(table, idx2d):
    mesh = plsc.VectorSubcoreMesh(core_axis_name="c", subcore_axis_name="s")
    return pl.kernel(
        _body,
        out_type=jax.ShapeDtypeStruct((TOTAL, DIM), jnp.float32),
        mesh=mesh,
        scratch_types=[
            pltpu.VMEM((NG, G), jnp.int32),
            pltpu.VMEM((GROUP_ROWS, DIM), jnp.float32),
            pltpu.VMEM((GROUP_ROWS, DIM), jnp.float32),
            pltpu.SemaphoreType.DMA,
            pltpu.SemaphoreType.DMA,
            pltpu.SemaphoreType.DMA,
            pltpu.SemaphoreType.DMA,
        ],
        compiler_params=pltpu.CompilerParams(use_tc_tiling_on_sc=False),
    )(table, idx2d)


def kernel(x, table):
    idx2d = x.reshape(TOTAL // G, G).astype(jnp.int32)
    out = _gather_flat(table, idx2d)
    return out.reshape(x.shape[0], x.shape[1], DIM)


# speedup vs baseline: 1.3641x; 1.2287x over previous
import jax
import jax.numpy as jnp
from jax import lax
from jax.experimental import pallas as pl
from jax.experimental.pallas import tpu as pltpu
from jax.experimental.pallas import tpu_sc as plsc

NUM_EMB = 1000000
DIM = 32
BATCH = 16384
HIST = 50

_info = plsc.get_sparse_core_info()
NC, NS = _info.num_cores, _info.num_subcores
NW = NC * NS
B_PER_W = BATCH // NW
G = 128
NGPL = B_PER_W // G


def _body(table_hbm, xt_hbm, out_hbm, idx_v, rows_v, tbuf, gsem, osem):
    wid = lax.axis_index("s") * NC + lax.axis_index("c")
    bbase = wid * B_PER_W
    lanes = lax.iota(jnp.int32, 16)

    def per_l(l, _):
        pltpu.sync_copy(xt_hbm.at[l, pl.ds(bbase, B_PER_W)], idx_v)
        descs = []
        for k in range(NGPL):
            descs.append(pltpu.async_copy(
                table_hbm.at[idx_v.at[pl.ds(k * G, G)]],
                rows_v.at[pl.ds(k * G, G)],
                gsem,
            ))
        for d in descs:
            d.wait()

        def tr(j0, _):
            row_idx = j0 * 16 + lanes
            for dd in range(DIM):
                col_idx = jnp.full((16,), dd, jnp.int32)
                v = plsc.load_gather(rows_v, [row_idx, col_idx])
                tbuf[dd, pl.ds(j0 * 16, 16)] = v
            return ()

        lax.fori_loop(0, B_PER_W // 16, tr, ())

        oc = pltpu.async_copy(
            tbuf, out_hbm.at[l].at[:, pl.ds(bbase, B_PER_W)], osem)
        oc.wait()
        return ()

    lax.fori_loop(0, HIST, per_l, ())


@jax.jit
def _gather_t(table, xt):
    mesh = plsc.VectorSubcoreMesh(core_axis_name="c", subcore_axis_name="s")
    return pl.kernel(
        _body,
        out_type=jax.ShapeDtypeStruct((HIST, DIM, BATCH), jnp.float32),
        mesh=mesh,
        scratch_types=[
            pltpu.VMEM((B_PER_W,), jnp.int32),
            pltpu.VMEM((B_PER_W, DIM), jnp.float32),
            pltpu.VMEM((DIM, B_PER_W), jnp.float32),
            pltpu.SemaphoreType.DMA,
            pltpu.SemaphoreType.DMA,
        ],
        compiler_params=pltpu.CompilerParams(
            use_tc_tiling_on_sc=False, needs_layout_passes=False),
    )(table, xt)


def kernel(x, table):
    xt = x.T.astype(jnp.int32)
    out_t = _gather_t(table, xt)
    return jnp.transpose(out_t, (2, 0, 1))


# speedup vs baseline: 1.5058x; 1.1039x over previous
import jax
import jax.numpy as jnp
from jax import lax
from jax.experimental import pallas as pl
from jax.experimental.pallas import tpu as pltpu
from jax.experimental.pallas import tpu_sc as plsc

NUM_EMB = 1000000
DIM = 32
BATCH = 16384
HIST = 50

_info = plsc.get_sparse_core_info()
NC, NS = _info.num_cores, _info.num_subcores
NW = NC * NS
B_PER_W = BATCH // NW
G = 128
NGPL = B_PER_W // G


def _body(table_hbm, xt_hbm, out_hbm, idx_all,
          rows0, rows1, tb0, tb1, gsem0, gsem1, osem0, osem1):
    wid = lax.axis_index("s") * NC + lax.axis_index("c")
    bbase = wid * B_PER_W
    lanes = lax.iota(jnp.int32, 16)
    cols = [jnp.full((16,), dd, jnp.int32) for dd in range(DIM)]

    pltpu.sync_copy(xt_hbm.at[:, pl.ds(bbase, B_PER_W)], idx_all)

    def fire(l, rows, sem):
        for k in range(NGPL):
            pltpu.async_copy(
                table_hbm.at[idx_all.at[l].at[pl.ds(k * G, G)]],
                rows.at[pl.ds(k * G, G)],
                sem,
            )

    def drain(rows, sem):
        for k in range(NGPL):
            pltpu.make_async_copy(
                table_hbm.at[idx_all.at[0].at[pl.ds(k * G, G)]],
                rows.at[pl.ds(k * G, G)],
                sem,
            ).wait()

    def transpose(rows, tb):
        def tr(j0, _):
            row_idx = j0 * 16 + lanes
            for dd in range(DIM):
                tb[dd, pl.ds(j0 * 16, 16)] = plsc.load_gather(
                    rows, [row_idx, cols[dd]])
            return ()
        lax.fori_loop(0, B_PER_W // 16, tr, ())

    def out_slice(l):
        return out_hbm.at[l].at[:, pl.ds(bbase, B_PER_W)]

    def wait_out(tb, sem):
        pltpu.make_async_copy(tb, out_slice(0), sem).wait()

    fire(0, rows0, gsem0)

    def step(i, _):
        l = 2 * i

        @pl.when(i > 0)
        def _():
            wait_out(tb0, osem0)
        fire(l + 1, rows1, gsem1)
        drain(rows0, gsem0)
        transpose(rows0, tb0)
        pltpu.async_copy(tb0, out_slice(l), osem0)

        @pl.when(i > 0)
        def _():
            wait_out(tb1, osem1)
        @pl.when(i < HIST // 2 - 1)
        def _():
            fire(l + 2, rows0, gsem0)
        drain(rows1, gsem1)
        transpose(rows1, tb1)
        pltpu.async_copy(tb1, out_slice(l + 1), osem1)
        return ()

    lax.fori_loop(0, HIST // 2, step, ())
    wait_out(tb0, osem0)
    wait_out(tb1, osem1)


@jax.jit
def _gather_t(table, xt):
    mesh = plsc.VectorSubcoreMesh(core_axis_name="c", subcore_axis_name="s")
    return pl.kernel(
        _body,
        out_type=jax.ShapeDtypeStruct((HIST, DIM, BATCH), jnp.float32),
        mesh=mesh,
        scratch_types=[
            pltpu.VMEM((HIST, B_PER_W), jnp.int32),
            pltpu.VMEM((B_PER_W, DIM), jnp.float32),
            pltpu.VMEM((B_PER_W, DIM), jnp.float32),
            pltpu.VMEM((DIM, B_PER_W), jnp.float32),
            pltpu.VMEM((DIM, B_PER_W), jnp.float32),
            pltpu.SemaphoreType.DMA,
            pltpu.SemaphoreType.DMA,
            pltpu.SemaphoreType.DMA,
            pltpu.SemaphoreType.DMA,
        ],
        compiler_params=pltpu.CompilerParams(
            use_tc_tiling_on_sc=False, needs_layout_passes=False),
    )(table, xt)


def kernel(x, table):
    xt = x.T.astype(jnp.int32)
    out_t = _gather_t(table, xt)
    return jnp.transpose(out_t, (2, 0, 1))
